# Initial kernel scaffold; baseline (speedup 1.0000x reference)
import jax
import jax.numpy as jnp
from jax.experimental import pallas as pl


def kernel(moe_hidden_states, mlp_hidden_states):
    raise NotImplementedError("write your pallas kernel here")


# speedup vs baseline: 2.8777x; 2.8777x over previous
import jax
import jax.numpy as jnp
from jax.experimental import pallas as pl

_INV_SQRT2 = 0.7071067811865476

_NUM_TOKENS = 16384
_HIDDEN = 4096
_BLOCK_T = 256


def _combine_body(moe_ref, mlp_ref, out_ref):
    out_ref[...] = (moe_ref[:, 0, :] + moe_ref[:, 1, :] + mlp_ref[...]) * _INV_SQRT2


def kernel(moe_hidden_states, mlp_hidden_states):
    n_tokens, combine_k, hidden = moe_hidden_states.shape
    block_t = _BLOCK_T
    grid = (n_tokens // block_t,)
    return pl.pallas_call(
        _combine_body,
        grid=grid,
        in_specs=[
            pl.BlockSpec((block_t, combine_k, hidden), lambda i: (i, 0, 0)),
            pl.BlockSpec((block_t, hidden), lambda i: (i, 0)),
        ],
        out_specs=pl.BlockSpec((block_t, hidden), lambda i: (i, 0)),
        out_shape=jax.ShapeDtypeStruct((n_tokens, hidden), jnp.float32),
    )(moe_hidden_states, mlp_hidden_states)
